# baseline (speedup 1.0000x reference)
import functools
import math

import jax
import jax.numpy as jnp
from jax import lax
from jax.experimental import pallas as pl
from jax.experimental.pallas import tpu as pltpu
import jax.experimental.pallas.tpu_sc as plsc

NC = 2
NS = 16
LANES = 16
CHUNK = 128
NBUF = 4
NPKT = 8


def _sc_segment_spmm(inph, pkt):
    N, D = inph.shape
    DH = D // NC
    K = pkt.shape[2]
    G = K // NBUF
    RZ = (N // NS) // 8 * 8
    TAIL = N - RZ * NS
    mesh = plsc.VectorSubcoreMesh(
        core_axis_name="c", subcore_axis_name="s", num_cores=NC)

    @functools.partial(
        pl.kernel,
        out_type=jax.ShapeDtypeStruct((NC, N, DH), jnp.float32),
        mesh=mesh,
        compiler_params=pltpu.CompilerParams(use_tc_tiling_on_sc=False),
        scratch_types=(
            [pltpu.VMEM_SHARED((N, DH), jnp.float32)]
            + [pltpu.VMEM_SHARED((N, DH), jnp.float32)]
            + [pltpu.VMEM((3, CHUNK), jnp.int32)] * NPKT
            + [pltpu.VMEM((CHUNK, DH), jnp.float32)] * NBUF
            + [pltpu.SemaphoreType.DMA] * (NPKT + 2 * NBUF)
        ),
    )
    def sc_kernel(inph_hbm, pkt_hbm, out_hbm, acc, inp_sh, *rest):
        pkts = rest[:NPKT]
        bufs = rest[NPKT:NPKT + NBUF]
        psem = rest[NPKT + NBUF:2 * NPKT + NBUF]
        gsem = rest[2 * NPKT + NBUF:2 * NPKT + 2 * NBUF]
        ssem = rest[2 * NPKT + 2 * NBUF:]
        c = lax.axis_index("c")
        s = lax.axis_index("s")

        def pkt_issue(j, slot):
            pltpu.async_copy(pkt_hbm.at[:, s, j], pkts[slot], psem[slot])

        def pkt_wait(j, slot):
            pltpu.make_async_copy(pkt_hbm.at[:, s, j], pkts[slot],
                                  psem[slot]).wait()

        def gather_issue(j, slot, b):
            pltpu.async_copy(inp_sh.at[pkts[slot].at[0]], bufs[b], gsem[b])

        def gather_wait(j, slot, b):
            pltpu.make_async_copy(inp_sh.at[pkts[slot].at[0]], bufs[b],
                                  gsem[b]).wait()

        def scatter_issue(slot, b):
            pltpu.async_copy(bufs[b], acc.at[pkts[slot].at[1]], ssem[b],
                             add=True)

        def scatter_wait(slot, b):
            pltpu.make_async_copy(bufs[b], acc.at[pkts[slot].at[1]],
                                  ssem[b]).wait()

        for j in range(NBUF):
            pkt_issue(j, j)

        stage_base = s * RZ
        col = c * DH
        stage_cp = pltpu.async_copy(
            inph_hbm.at[pl.ds(stage_base, RZ), pl.ds(col, DH)],
            inp_sh.at[pl.ds(stage_base, RZ)], gsem[2])
        if TAIL:
            @pl.when(s == NS - 1)
            def _stage_tail():
                pltpu.sync_copy(
                    inph_hbm.at[pl.ds(RZ * NS, TAIL), pl.ds(col, DH)],
                    inp_sh.at[pl.ds(RZ * NS, TAIL)])

        zb = bufs[0]

        def zrow(e, carry):
            for d in range(DH // LANES):
                zb[e, pl.ds(d * LANES, LANES)] = jnp.zeros((LANES,),
                                                           jnp.float32)
            return carry

        lax.fori_loop(0, CHUNK, zrow, 0)
        zbase = s * RZ
        nfull = RZ // CHUNK
        for i in range(nfull):
            pltpu.sync_copy(zb, acc.at[pl.ds(zbase + i * CHUNK, CHUNK)])
        rem = RZ - nfull * CHUNK
        if rem:
            pltpu.sync_copy(zb.at[pl.ds(0, rem)],
                            acc.at[pl.ds(zbase + nfull * CHUNK, rem)])
        if TAIL:
            @pl.when(s == NS - 1)
            def _zero_tail():
                pltpu.sync_copy(zb.at[pl.ds(0, TAIL)],
                                acc.at[pl.ds(RZ * NS, TAIL)])
        stage_cp.wait()
        plsc.subcore_barrier()

        for j in range(2):
            pkt_wait(j, j)
            gather_issue(j, j, j)

        G2 = G // 2

        def group(gg, carry):
            for p in range(2):
                for b in range(NBUF):
                    jj = (gg * 2 + p) * NBUF + b
                    m8 = (4 * p + b) % NPKT
                    pj = m8
                    pn = (m8 + 2) % NPKT
                    pf = (m8 + NBUF) % NPKT
                    pm2 = (m8 - 2) % NPKT
                    bn = (b + 2) % NBUF

                    if p == 0:
                        pkt_issue(jj + NBUF, pf)
                    else:
                        pl.when(gg < G2 - 1)(
                            lambda: pkt_issue(jj + NBUF, pf))

                    def buf_prefetch():
                        pkt_wait(jj + 2, pn)
                        gather_issue(jj + 2, pn, bn)

                    def wait_and_prefetch():
                        scatter_wait(pm2, bn)
                        buf_prefetch()

                    if b < 2:
                        if p == 0:
                            pl.when(gg == 0)(buf_prefetch)
                            pl.when(gg >= 1)(wait_and_prefetch)
                        else:
                            wait_and_prefetch()
                    else:
                        if p == 0:
                            wait_and_prefetch()
                        else:
                            pl.when(gg < G2 - 1)(wait_and_prefetch)

                    gather_wait(jj, pj, b)
                    buf = bufs[b]
                    vrow = pkts[pj]
                    def scale(g2, carry2):
                        v16i = vrow[2, pl.ds(g2 * LANES, LANES)]
                        v16 = lax.bitcast_convert_type(v16i, jnp.float32)
                        for e in range(LANES):
                            v = v16[e]
                            row = g2 * LANES + e
                            for d in range(DH // LANES):
                                sl = pl.ds(d * LANES, LANES)
                                buf[row, sl] = buf[row, sl] * v
                        return carry2

                    lax.fori_loop(0, CHUNK // LANES, scale, 0, unroll=8)
                    scatter_issue(pj, b)
            return carry

        lax.fori_loop(0, G2, group, 0)

        for b in range(NBUF):
            jl = (G - 1) * NBUF + b
            scatter_wait(jl % NPKT, b)
        plsc.subcore_barrier()
        pltpu.sync_copy(acc.at[pl.ds(zbase, RZ)],
                        out_hbm.at[c, pl.ds(zbase, RZ)])
        if TAIL:
            @pl.when(s == NS - 1)
            def _write_tail():
                pltpu.sync_copy(acc.at[pl.ds(RZ * NS, TAIL)],
                                out_hbm.at[c, pl.ds(RZ * NS, TAIL)])

    return sc_kernel(inph, pkt)


def _tc_combine(p0, p1, feature, weight, scal):
    N, D = feature.shape
    BR = 1000
    nb = N // BR

    def body(scal_ref, p0_ref, p1_ref, f_ref, w_ref, o_ref):
        a = scal_ref[0]
        bt = scal_ref[1]
        hi = jnp.concatenate([p0_ref[...], p1_ref[...]], axis=1)
        sup = (1.0 - a) * hi + a * f_ref[...]
        o_ref[...] = (1.0 - bt) * sup + bt * jnp.dot(
            sup, w_ref[...], preferred_element_type=jnp.float32)

    return pl.pallas_call(
        body,
        grid=(nb,),
        in_specs=[
            pl.BlockSpec(memory_space=pltpu.SMEM),
            pl.BlockSpec((BR, D // 2), lambda i: (i, 0)),
            pl.BlockSpec((BR, D // 2), lambda i: (i, 0)),
            pl.BlockSpec((BR, D), lambda i: (i, 0)),
            pl.BlockSpec((D, D), lambda i: (0, 0)),
        ],
        out_specs=pl.BlockSpec((BR, D), lambda i: (i, 0)),
        out_shape=jax.ShapeDtypeStruct((N, D), jnp.float32),
    )(scal, p0, p1, feature, weight)


def kernel(feature, input, adj_indices, adj_values, weight, alpha, lamda, l):
    N, D = input.shape
    E = adj_values.shape[0]
    DH = D // NC
    beta = jnp.log(lamda / l + 1)

    K = math.ceil(E / (NS * CHUNK))
    K = ((K + 2 * NBUF - 1) // (2 * NBUF)) * (2 * NBUF)
    EP = NS * CHUNK * K
    pad = EP - E

    src = jnp.pad(adj_indices[0], (0, pad)).reshape(NS, K, CHUNK)
    dst = jnp.pad(adj_indices[1], (0, pad)).reshape(NS, K, CHUNK)
    val = lax.bitcast_convert_type(jnp.pad(adj_values, (0, pad)),
                                   jnp.int32).reshape(NS, K, CHUNK)
    pkt = jnp.stack([src, dst, val], axis=0)

    parts = _sc_segment_spmm(input, pkt)
    scal = jnp.stack([jnp.float32(alpha), jnp.float32(beta)])
    return _tc_combine(parts[0], parts[1], feature, weight, scal)


# speedup vs baseline: 1.1990x; 1.1990x over previous
import functools
import math

import jax
import jax.numpy as jnp
from jax import lax
from jax.experimental import pallas as pl
from jax.experimental.pallas import tpu as pltpu
import jax.experimental.pallas.tpu_sc as plsc

NC = 2
NS = 16
LANES = 16
CHUNK = 128
NBUF = 4
NPKT = 8


def _sc_segment_spmm(inph, pkt):
    N, D = inph.shape
    DH = D // NC
    K = pkt.shape[2]
    G = K // NBUF
    RZ = (N // NS) // 8 * 8
    TAIL = N - RZ * NS
    mesh = plsc.VectorSubcoreMesh(
        core_axis_name="c", subcore_axis_name="s", num_cores=NC)

    @functools.partial(
        pl.kernel,
        out_type=jax.ShapeDtypeStruct((NC, N, DH), jnp.float32),
        mesh=mesh,
        compiler_params=pltpu.CompilerParams(use_tc_tiling_on_sc=False),
        scratch_types=(
            [pltpu.VMEM_SHARED((N, DH), jnp.float32)]
            + [pltpu.VMEM_SHARED((N, DH), jnp.float32)]
            + [pltpu.VMEM((3, CHUNK), jnp.int32)] * NPKT
            + [pltpu.VMEM((CHUNK, DH), jnp.float32)] * NBUF
            + [pltpu.SemaphoreType.DMA] * (NPKT + 2 * NBUF)
        ),
    )
    def sc_kernel(inph_hbm, pkt_hbm, out_hbm, acc, inp_sh, *rest):
        pkts = rest[:NPKT]
        bufs = rest[NPKT:NPKT + NBUF]
        psem = rest[NPKT + NBUF:2 * NPKT + NBUF]
        gsem = rest[2 * NPKT + NBUF:2 * NPKT + 2 * NBUF]
        ssem = rest[2 * NPKT + 2 * NBUF:]
        c = lax.axis_index("c")
        s = lax.axis_index("s")

        def pkt_issue(j, slot):
            pltpu.async_copy(pkt_hbm.at[:, s, j], pkts[slot], psem[slot])

        def pkt_wait(j, slot):
            pltpu.make_async_copy(pkt_hbm.at[:, s, j], pkts[slot],
                                  psem[slot]).wait()

        def gather_issue(j, slot, b):
            pltpu.async_copy(inp_sh.at[pkts[slot].at[0]], bufs[b], gsem[b])

        def gather_wait(j, slot, b):
            pltpu.make_async_copy(inp_sh.at[pkts[slot].at[0]], bufs[b],
                                  gsem[b]).wait()

        def scatter_issue(slot, b):
            pltpu.async_copy(bufs[b], acc.at[pkts[slot].at[1]], ssem[b],
                             add=True)

        def scatter_wait(slot, b):
            pltpu.make_async_copy(bufs[b], acc.at[pkts[slot].at[1]],
                                  ssem[b]).wait()

        for j in range(NBUF):
            pkt_issue(j, j)

        stage_base = s * RZ
        col = c * DH
        stage_cp = pltpu.async_copy(
            inph_hbm.at[pl.ds(stage_base, RZ), pl.ds(col, DH)],
            inp_sh.at[pl.ds(stage_base, RZ)], gsem[2])
        if TAIL:
            @pl.when(s == NS - 1)
            def _stage_tail():
                pltpu.sync_copy(
                    inph_hbm.at[pl.ds(RZ * NS, TAIL), pl.ds(col, DH)],
                    inp_sh.at[pl.ds(RZ * NS, TAIL)])

        zb = bufs[0]

        def zrow(e, carry):
            for d in range(DH // LANES):
                zb[e, pl.ds(d * LANES, LANES)] = jnp.zeros((LANES,),
                                                           jnp.float32)
            return carry

        lax.fori_loop(0, CHUNK, zrow, 0)
        zbase = s * RZ
        nfull = RZ // CHUNK
        for i in range(nfull):
            pltpu.sync_copy(zb, acc.at[pl.ds(zbase + i * CHUNK, CHUNK)])
        rem = RZ - nfull * CHUNK
        if rem:
            pltpu.sync_copy(zb.at[pl.ds(0, rem)],
                            acc.at[pl.ds(zbase + nfull * CHUNK, rem)])
        if TAIL:
            @pl.when(s == NS - 1)
            def _zero_tail():
                pltpu.sync_copy(zb.at[pl.ds(0, TAIL)],
                                acc.at[pl.ds(RZ * NS, TAIL)])
        stage_cp.wait()
        plsc.subcore_barrier()

        for j in range(2):
            pkt_wait(j, j)
            gather_issue(j, j, j)

        G2 = G // 2

        def group(gg, carry):
            for p in range(2):
                for b in range(NBUF):
                    jj = (gg * 2 + p) * NBUF + b
                    m8 = (4 * p + b) % NPKT
                    pj = m8
                    pn = (m8 + 2) % NPKT
                    pf = (m8 + NBUF) % NPKT
                    pm2 = (m8 - 2) % NPKT
                    bn = (b + 2) % NBUF

                    if p == 0:
                        pkt_issue(jj + NBUF, pf)
                    else:
                        pl.when(gg < G2 - 1)(
                            lambda: pkt_issue(jj + NBUF, pf))

                    def buf_prefetch():
                        pkt_wait(jj + 2, pn)
                        gather_issue(jj + 2, pn, bn)

                    def wait_and_prefetch():
                        scatter_wait(pm2, bn)
                        buf_prefetch()

                    if b < 2:
                        if p == 0:
                            pl.when(gg == 0)(buf_prefetch)
                            pl.when(gg >= 1)(wait_and_prefetch)
                        else:
                            wait_and_prefetch()
                    else:
                        if p == 0:
                            wait_and_prefetch()
                        else:
                            pl.when(gg < G2 - 1)(wait_and_prefetch)

                    gather_wait(jj, pj, b)
                    buf = bufs[b]
                    vrow = pkts[pj]
                    def scale(g2, carry2):
                        v16i = vrow[2, pl.ds(g2 * LANES, LANES)]
                        v16 = lax.bitcast_convert_type(v16i, jnp.float32)
                        for e in range(LANES):
                            v = v16[e]
                            row = g2 * LANES + e
                            for d in range(DH // LANES):
                                sl = pl.ds(d * LANES, LANES)
                                buf[row, sl] = buf[row, sl] * v
                        return carry2

                    lax.fori_loop(0, CHUNK // LANES, scale, 0, unroll=4)
                    scatter_issue(pj, b)
            return carry

        lax.fori_loop(0, G2, group, 0)

        for b in range(NBUF):
            jl = (G - 1) * NBUF + b
            scatter_wait(jl % NPKT, b)
        plsc.subcore_barrier()
        pltpu.sync_copy(acc.at[pl.ds(zbase, RZ)],
                        out_hbm.at[c, pl.ds(zbase, RZ)])
        if TAIL:
            @pl.when(s == NS - 1)
            def _write_tail():
                pltpu.sync_copy(acc.at[pl.ds(RZ * NS, TAIL)],
                                out_hbm.at[c, pl.ds(RZ * NS, TAIL)])

    return sc_kernel(inph, pkt)


def _tc_combine(p0, p1, feature, weight, scal):
    N, D = feature.shape
    BR = 1000
    nb = N // BR

    def body(scal_ref, p0_ref, p1_ref, f_ref, w_ref, o_ref):
        a = scal_ref[0]
        bt = scal_ref[1]
        hi = jnp.concatenate([p0_ref[...], p1_ref[...]], axis=1)
        sup = (1.0 - a) * hi + a * f_ref[...]
        o_ref[...] = (1.0 - bt) * sup + bt * jnp.dot(
            sup, w_ref[...], preferred_element_type=jnp.float32)

    return pl.pallas_call(
        body,
        grid=(nb,),
        in_specs=[
            pl.BlockSpec(memory_space=pltpu.SMEM),
            pl.BlockSpec((BR, D // 2), lambda i: (i, 0)),
            pl.BlockSpec((BR, D // 2), lambda i: (i, 0)),
            pl.BlockSpec((BR, D), lambda i: (i, 0)),
            pl.BlockSpec((D, D), lambda i: (0, 0)),
        ],
        out_specs=pl.BlockSpec((BR, D), lambda i: (i, 0)),
        out_shape=jax.ShapeDtypeStruct((N, D), jnp.float32),
    )(scal, p0, p1, feature, weight)


def kernel(feature, input, adj_indices, adj_values, weight, alpha, lamda, l):
    N, D = input.shape
    E = adj_values.shape[0]
    DH = D // NC
    beta = jnp.log(lamda / l + 1)

    K = math.ceil(E / (NS * CHUNK))
    K = ((K + 2 * NBUF - 1) // (2 * NBUF)) * (2 * NBUF)
    EP = NS * CHUNK * K
    pad = EP - E

    src = jnp.pad(adj_indices[0], (0, pad)).reshape(NS, K, CHUNK)
    dst = jnp.pad(adj_indices[1], (0, pad)).reshape(NS, K, CHUNK)
    val = lax.bitcast_convert_type(jnp.pad(adj_values, (0, pad)),
                                   jnp.int32).reshape(NS, K, CHUNK)
    pkt = jnp.stack([src, dst, val], axis=0)

    parts = _sc_segment_spmm(input, pkt)
    scal = jnp.stack([jnp.float32(alpha), jnp.float32(beta)])
    return _tc_combine(parts[0], parts[1], feature, weight, scal)
